# baseline (speedup 1.0000x reference)
import math

import jax
import jax.numpy as jnp
from jax.experimental import pallas as pl
from jax.experimental.pallas import tpu as pltpu

_SIZE = 100000
_SMOOTH = 0.1
_CONF = 1.0 - _SMOOTH
_FILL = _SMOOTH / (_SIZE - 2)
_N = 1024

_C0 = float(
    _N * (_SIZE - 1) * _FILL * math.log(_FILL) + _N * _CONF * math.log(_CONF)
)

_ROWS_PER_BLK = 32
_GRID = _N // _ROWS_PER_BLK


def _body(t_ref, x_ref, o_ref, acc_ref):
    step = pl.program_id(0)

    @pl.when(step == 0)
    def _init():
        acc_ref[0] = 0.0
        acc_ref[1] = 0.0

    xb = x_ref[...]
    t = t_ref[0, 0, :]
    col = jax.lax.broadcasted_iota(jnp.int32, xb.shape, 1)
    mask = col == t[:, None]
    acc_ref[0] += jnp.sum(xb)
    acc_ref[1] += jnp.sum(jnp.where(mask, xb, 0.0))

    @pl.when(step == _GRID - 1)
    def _fin():
        val = (
            jnp.float32(_C0)
            - jnp.float32(_FILL) * acc_ref[0]
            + jnp.float32(_FILL - _CONF) * acc_ref[1]
        )
        o_ref[...] = val[None, None]


def kernel(x, target):
    t3 = target.reshape(_GRID, 1, _ROWS_PER_BLK)
    out = pl.pallas_call(
        _body,
        grid=(_GRID,),
        in_specs=[
            pl.BlockSpec((1, 1, _ROWS_PER_BLK), lambda i: (i, 0, 0)),
            pl.BlockSpec((_ROWS_PER_BLK, _SIZE), lambda i: (i, 0)),
        ],
        out_specs=pl.BlockSpec((1, 1), lambda i: (0, 0)),
        out_shape=jax.ShapeDtypeStruct((1, 1), jnp.float32),
        scratch_shapes=[pltpu.SMEM((2,), jnp.float32)],
        compiler_params=pltpu.CompilerParams(
            dimension_semantics=("arbitrary",),
        ),
    )(t3, x)
    return out[0, 0]


# speedup vs baseline: 1.0569x; 1.0569x over previous
import math

import jax
import jax.numpy as jnp
from jax.experimental import pallas as pl
from jax.experimental.pallas import tpu as pltpu

_SIZE = 100000
_SMOOTH = 0.1
_CONF = 1.0 - _SMOOTH
_FILL = _SMOOTH / (_SIZE - 2)
_N = 1024

_C0 = float(
    _N * (_SIZE - 1) * _FILL * math.log(_FILL) + _N * _CONF * math.log(_CONF)
)

_NSTREAM = 4
_ROWS_PER_BLK = 16
_GRID = _N // (_ROWS_PER_BLK * _NSTREAM)


def _body(*refs):
    t_refs = refs[:_NSTREAM]
    x_refs = refs[_NSTREAM:2 * _NSTREAM]
    o_ref = refs[2 * _NSTREAM]
    acc_ref = refs[2 * _NSTREAM + 1]
    step = pl.program_id(0)

    @pl.when(step == 0)
    def _init():
        acc_ref[0] = 0.0
        acc_ref[1] = 0.0

    s0 = jnp.float32(0.0)
    s1 = jnp.float32(0.0)
    for k in range(_NSTREAM):
        xb = x_refs[k][...]
        t = t_refs[k][0, 0, :]
        col = jax.lax.broadcasted_iota(jnp.int32, xb.shape, 1)
        mask = col == t[:, None]
        s0 += jnp.sum(xb)
        s1 += jnp.sum(jnp.where(mask, xb, 0.0))
    acc_ref[0] += s0
    acc_ref[1] += s1

    @pl.when(step == _GRID - 1)
    def _fin():
        val = (
            jnp.float32(_C0)
            - jnp.float32(_FILL) * acc_ref[0]
            + jnp.float32(_FILL - _CONF) * acc_ref[1]
        )
        o_ref[...] = val[None, None]


def kernel(x, target):
    nblk = _N // _ROWS_PER_BLK
    t3 = target.reshape(nblk, 1, _ROWS_PER_BLK)
    t_specs = [
        pl.BlockSpec(
            (1, 1, _ROWS_PER_BLK),
            (lambda k: (lambda i: (i + k * _GRID, 0, 0)))(k),
        )
        for k in range(_NSTREAM)
    ]
    x_specs = [
        pl.BlockSpec(
            (_ROWS_PER_BLK, _SIZE),
            (lambda k: (lambda i: (i + k * _GRID, 0)))(k),
        )
        for k in range(_NSTREAM)
    ]
    out = pl.pallas_call(
        _body,
        grid=(_GRID,),
        in_specs=t_specs + x_specs,
        out_specs=pl.BlockSpec((1, 1), lambda i: (0, 0)),
        out_shape=jax.ShapeDtypeStruct((1, 1), jnp.float32),
        scratch_shapes=[pltpu.SMEM((2,), jnp.float32)],
        compiler_params=pltpu.CompilerParams(
            dimension_semantics=("arbitrary",),
        ),
    )(*([t3] * _NSTREAM + [x] * _NSTREAM))
    return out[0, 0]
